# baseline (speedup 1.0000x reference)
import jax
import jax.numpy as jnp
from jax.experimental import pallas as pl
from jax.experimental.pallas import tpu as pltpu

_BV = 2000


def _proj_block(x_ref, w_ref, b_ref, o_ref):
    acc = jax.lax.dot_general(
        w_ref[...],
        x_ref[...],
        dimension_numbers=(((1,), (1,)), ((), ())),
        preferred_element_type=jnp.float32,
    )
    o_ref[...] = acc + b_ref[...]


def _logits_t(inputs, W, b):
    batch, nhid = inputs.shape
    ntokens = W.shape[0]
    b2 = b.reshape(ntokens, 1)
    grid = (ntokens // _BV,)
    return pl.pallas_call(
        _proj_block,
        grid=grid,
        in_specs=[
            pl.BlockSpec((batch, nhid), lambda i: (0, 0)),
            pl.BlockSpec((_BV, nhid), lambda i: (i, 0)),
            pl.BlockSpec((_BV, 1), lambda i: (i, 0)),
        ],
        out_specs=pl.BlockSpec((_BV, batch), lambda i: (i, 0)),
        out_shape=jax.ShapeDtypeStruct((ntokens, batch), jnp.float32),
        compiler_params=pltpu.CompilerParams(
            dimension_semantics=("arbitrary",),
        ),
    )(inputs, W, b2)


def kernel(inputs, labels, W, b):
    return (_logits_t(inputs, W, b).T, labels)


# speedup vs baseline: 1.0242x; 1.0242x over previous
import jax
import jax.numpy as jnp
from jax.experimental import pallas as pl
from jax.experimental.pallas import tpu as pltpu

_BV = 5000


def _proj_block(x_ref, w_ref, b_ref, o_ref):
    acc = jax.lax.dot_general(
        w_ref[...],
        x_ref[...],
        dimension_numbers=(((1,), (1,)), ((), ())),
        preferred_element_type=jnp.float32,
    )
    o_ref[...] = acc + b_ref[...]


def _logits_t(inputs, W, b):
    batch, nhid = inputs.shape
    ntokens = W.shape[0]
    b2 = b.reshape(ntokens, 1)
    grid = (ntokens // _BV,)
    return pl.pallas_call(
        _proj_block,
        grid=grid,
        in_specs=[
            pl.BlockSpec((batch, nhid), lambda i: (0, 0)),
            pl.BlockSpec((_BV, nhid), lambda i: (i, 0)),
            pl.BlockSpec((_BV, 1), lambda i: (i, 0)),
        ],
        out_specs=pl.BlockSpec((_BV, batch), lambda i: (i, 0)),
        out_shape=jax.ShapeDtypeStruct((ntokens, batch), jnp.float32),
        compiler_params=pltpu.CompilerParams(
            dimension_semantics=("arbitrary",),
        ),
    )(inputs, W, b2)


def kernel(inputs, labels, W, b):
    return (_logits_t(inputs, W, b).T, labels)
